# baseline (speedup 1.0000x reference)
import functools

import jax
import jax.numpy as jnp
from jax import lax
from jax.experimental import pallas as pl
from jax.experimental.pallas import tpu as pltpu
from jax.experimental.pallas import tpu_sc as plsc

_NUM_CODEBOOKS = 4
_SUB_VOCAB = 250000
_HIDDEN = 64
_BATCH = 16384

_NW = 32
_TOTAL = _NUM_CODEBOOKS * _BATCH
_PER_W = _TOTAL // _NW
_NBLK = 4
_WB = _PER_W // _NBLK
_G = 128
_GROUPS = _WB // _G

_mesh = plsc.VectorSubcoreMesh(core_axis_name="c", subcore_axis_name="s")


@functools.partial(
    pl.kernel,
    out_type=jax.ShapeDtypeStruct((_TOTAL, 2 * _HIDDEN), jnp.float32),
    mesh=_mesh,
    scratch_types=[
        pltpu.VMEM((8, _G), jnp.int32),
        pltpu.VMEM((_WB, 2 * _HIDDEN), jnp.float32),
        pltpu.SemaphoreType.DMA,
    ],
)
def _gather_wide(table_hbm, idx_hbm, out_hbm, idx_v, rows_v, sem):
    wid = lax.axis_index("s") * 2 + lax.axis_index("c")
    for k in range(_NBLK):
        row = wid * _NBLK + k
        pltpu.sync_copy(idx_hbm.at[row], idx_v)
        copies = []
        for g in range(_GROUPS):
            dst = rows_v.at[pl.ds(g * _G, _G)]
            copies.append(pltpu.async_copy(table_hbm.at[idx_v.at[g]], dst, sem))
        for cpy in copies:
            cpy.wait()
        base = wid * _PER_W + k * _WB
        pltpu.sync_copy(rows_v, out_hbm.at[pl.ds(base, _WB)])


_TOTAL_ROWS_W = _NUM_CODEBOOKS * _SUB_VOCAB // 2

_VB = 2000
_RBLK = _SUB_VOCAB // _VB


def _repack_body(t_hbm, w_ref, buf, sems):
    c = pl.program_id(0)
    j = pl.program_id(1)
    i = c * _RBLK + j
    slot = lax.rem(i, 2)

    def _start(step, slot):
        c2 = lax.div(step, _RBLK)
        j2 = lax.rem(step, _RBLK)
        pltpu.make_async_copy(
            t_hbm.at[c2, :, pl.ds(j2 * _VB, _VB)], buf.at[slot],
            sems.at[slot]).start()

    @pl.when(i == 0)
    def _():
        _start(0, 0)

    @pl.when(i + 1 < _NUM_CODEBOOKS * _RBLK)
    def _():
        _start(i + 1, lax.rem(i + 1, 2))

    pltpu.make_async_copy(
        t_hbm.at[c, :, pl.ds(j * _VB, _VB)], buf.at[slot],
        sems.at[slot]).wait()
    x = buf[slot]
    w_ref[...] = x.T.reshape(_VB // 2, 2 * _HIDDEN)


_repack = pl.pallas_call(
    _repack_body,
    out_shape=jax.ShapeDtypeStruct((_TOTAL_ROWS_W, 2 * _HIDDEN), jnp.float32),
    grid=(_NUM_CODEBOOKS, _RBLK),
    in_specs=[pl.BlockSpec(memory_space=pl.ANY)],
    out_specs=pl.BlockSpec((_VB // 2, 2 * _HIDDEN),
                           lambda c, j: (c * _RBLK + j, 0)),
    scratch_shapes=[
        pltpu.VMEM((2, _HIDDEN, _VB), jnp.float32),
        pltpu.SemaphoreType.DMA((2,)),
    ],
)


_TBLK = 512


def _fold_body(g_ref, p_ref, o_ref):
    g = g_ref[...]
    p = p_ref[...]
    sel = jnp.where(p[:, :, None] == 1, g[:, :, _HIDDEN:], g[:, :, :_HIDDEN])
    o_ref[...] = jnp.sum(sel, axis=0)


_fold = pl.pallas_call(
    _fold_body,
    out_shape=jax.ShapeDtypeStruct((_BATCH, _HIDDEN), jnp.float32),
    grid=(_BATCH // _TBLK,),
    in_specs=[
        pl.BlockSpec((_NUM_CODEBOOKS, _TBLK, 2 * _HIDDEN), lambda i: (0, i, 0)),
        pl.BlockSpec((_NUM_CODEBOOKS, _TBLK), lambda i: (0, i)),
    ],
    out_specs=pl.BlockSpec((_TBLK, _HIDDEN), lambda i: (i, 0)),
)


def kernel(indices, codebooks):
    table = jnp.concatenate(
        [codebooks[c].reshape(_SUB_VOCAB // 2, 2 * _HIDDEN)
         for c in range(_NUM_CODEBOOKS)], axis=0)
    offs = (jnp.arange(_NUM_CODEBOOKS, dtype=jnp.int32) * _SUB_VOCAB)[None, :]
    flat = indices + offs
    parity = (flat & 1).T
    wide = (flat >> 1).T.reshape(_TOTAL)
    idx_arr = jnp.pad(wide.reshape(_NW * _NBLK, _GROUPS, _G),
                      ((0, 0), (0, 8 - _GROUPS), (0, 0)))
    gathered = _gather_wide(table, idx_arr)
    g3 = gathered.reshape(_NUM_CODEBOOKS, _BATCH, 2 * _HIDDEN)
    return _fold(g3, parity)


# speedup vs baseline: 1.3736x; 1.3736x over previous
import functools

import jax
import jax.numpy as jnp
from jax import lax
from jax.experimental import pallas as pl
from jax.experimental.pallas import tpu as pltpu
from jax.experimental.pallas import tpu_sc as plsc

_NUM_CODEBOOKS = 4
_SUB_VOCAB = 250000
_HIDDEN = 64
_BATCH = 16384

_NW = 32
_TOTAL = _NUM_CODEBOOKS * _BATCH
_PER_W = _TOTAL // _NW
_NBLK = 4
_WB = _PER_W // _NBLK
_G = 128
_GROUPS = _WB // _G

_mesh = plsc.VectorSubcoreMesh(core_axis_name="c", subcore_axis_name="s")


@functools.partial(
    pl.kernel,
    out_type=jax.ShapeDtypeStruct((_TOTAL, 2 * _HIDDEN), jnp.float32),
    mesh=_mesh,
    scratch_types=[
        pltpu.VMEM((8, _G), jnp.int32),
        pltpu.VMEM((_WB, 2 * _HIDDEN), jnp.float32),
        pltpu.SemaphoreType.DMA,
    ],
)
def _gather_wide(table_hbm, idx_hbm, out_hbm, idx_v, rows_v, sem):
    wid = lax.axis_index("s") * 2 + lax.axis_index("c")
    for k in range(_NBLK):
        row = wid * _NBLK + k
        pltpu.sync_copy(idx_hbm.at[row], idx_v)
        copies = []
        for g in range(_GROUPS):
            dst = rows_v.at[pl.ds(g * _G, _G)]
            copies.append(pltpu.async_copy(table_hbm.at[idx_v.at[g]], dst, sem))
        for cpy in copies:
            cpy.wait()
        base = wid * _PER_W + k * _WB
        pltpu.sync_copy(rows_v, out_hbm.at[pl.ds(base, _WB)])


_TOTAL_ROWS_W = _NUM_CODEBOOKS * _SUB_VOCAB // 2

_VB = 1000
_RBLK = _SUB_VOCAB // 2 // _VB


_HALF_V = _SUB_VOCAB // 2


def _repack_body(a_ref, b_ref, w_ref):
    w_ref[...] = jnp.concatenate([a_ref[0], b_ref[0]], axis=1)


_repack = pl.pallas_call(
    _repack_body,
    out_shape=jax.ShapeDtypeStruct((_TOTAL_ROWS_W, 2 * _HIDDEN), jnp.float32),
    grid=(_NUM_CODEBOOKS, _RBLK),
    in_specs=[
        pl.BlockSpec((1, _VB, _HIDDEN), lambda c, j: (c, j, 0)),
        pl.BlockSpec((1, _VB, _HIDDEN), lambda c, j: (c, j + _RBLK, 0)),
    ],
    out_specs=pl.BlockSpec((_VB, 2 * _HIDDEN),
                           lambda c, j: (c * _RBLK + j, 0)),
)


_TBLK = 512


def _fold_body(g_ref, p_ref, o_ref):
    g = g_ref[...]
    p = p_ref[...]
    sel = jnp.where(p[:, :, None] == 1, g[:, :, _HIDDEN:], g[:, :, :_HIDDEN])
    o_ref[...] = jnp.sum(sel, axis=0)


_fold = pl.pallas_call(
    _fold_body,
    out_shape=jax.ShapeDtypeStruct((_BATCH, _HIDDEN), jnp.float32),
    grid=(_BATCH // _TBLK,),
    in_specs=[
        pl.BlockSpec((_NUM_CODEBOOKS, _TBLK, 2 * _HIDDEN), lambda i: (0, i, 0)),
        pl.BlockSpec((_NUM_CODEBOOKS, _TBLK), lambda i: (0, i)),
    ],
    out_specs=pl.BlockSpec((_TBLK, _HIDDEN), lambda i: (i, 0)),
)


def kernel(indices, codebooks):
    table = _repack(codebooks, codebooks)
    offs = (jnp.arange(_NUM_CODEBOOKS, dtype=jnp.int32) * _HALF_V)[None, :]
    parity = (indices // _HALF_V).T
    wide = ((indices % _HALF_V) + offs).T.reshape(_TOTAL)
    idx_arr = jnp.pad(wide.reshape(_NW * _NBLK, _GROUPS, _G),
                      ((0, 0), (0, 8 - _GROUPS), (0, 0)))
    gathered = _gather_wide(table, idx_arr)
    g3 = gathered.reshape(_NUM_CODEBOOKS, _BATCH, 2 * _HIDDEN)
    return _fold(g3, parity)
